# baseline (speedup 1.0000x reference)
import jax
import jax.numpy as jnp
from jax import lax
from jax.experimental import pallas as pl
from jax.experimental.pallas import tpu as pltpu
from jax.experimental.pallas import tpu_sc as plsc

T = 2048
H = 1024
F = 4096
E = 8
BT = 256
NBLK = T * 2 // BT + E
PAD = NBLK * BT

_mesh = plsc.VectorSubcoreMesh(core_axis_name="c", subcore_axis_name="s")
_sc_params = pltpu.CompilerParams(needs_layout_passes=False)


def _router_body(flat_ref, wr_ref, i1_ref, i2_ref, wa_ref, wb_ref):
    l = lax.dot_general(flat_ref[...], wr_ref[...],
                        (((1,), (1,)), ((), ())),
                        preferred_element_type=jnp.float32)
    lane = lax.broadcasted_iota(jnp.int32, (T, E), 1)
    m1 = jnp.max(l, axis=1, keepdims=True)
    i1 = jnp.min(jnp.where(l >= m1, lane, E), axis=1, keepdims=True)
    l2 = jnp.where(lane == i1, jnp.float32(-1e30), l)
    m2 = jnp.max(l2, axis=1, keepdims=True)
    i2 = jnp.min(jnp.where(l2 >= m2, lane, E), axis=1, keepdims=True)
    wa = 1.0 / (1.0 + jnp.exp(m2 - m1))
    i1_ref[...] = i1
    i2_ref[...] = i2
    wa_ref[...] = wa
    wb_ref[...] = 1.0 - wa


def _run_router(flat, Wr):
    return pl.pallas_call(
        _router_body,
        out_shape=[
            jax.ShapeDtypeStruct((T, 1), jnp.int32),
            jax.ShapeDtypeStruct((T, 1), jnp.int32),
            jax.ShapeDtypeStruct((T, 1), jnp.float32),
            jax.ShapeDtypeStruct((T, 1), jnp.float32),
        ],
    )(flat, Wr)


def _dispatch_body(top1_hbm, top2_hbm, wa_hbm, wb_hbm,
                   stok_hbm, sw_hbm, p0_hbm, p1_hbm, btab_hbm,
                   id1_v, id2_v, wa_v, wb_v, seg_tok_v, seg_w_v, part_v,
                   btab_v, ztok_v, zw_v):
    c = lax.axis_index("c")
    s = lax.axis_index("s")

    @pl.when(jnp.logical_and(c == 0, s < 10))
    def _work():
        pltpu.sync_copy(top1_hbm, id1_v)
        pltpu.sync_copy(top2_hbm, id2_v)
        pltpu.sync_copy(wa_hbm, wa_v)
        pltpu.sync_copy(wb_hbm, wb_v)

        def cnt_body(j, acc):
            v1 = id1_v[pl.ds(j * 16, 16)]
            v2 = id2_v[pl.ds(j * 16, 16)]
            return tuple(acc[e]
                         + (v1 == e).astype(jnp.int32)
                         + (v2 == e).astype(jnp.int32)
                         for e in range(E))

        accs = lax.fori_loop(0, T // 16, cnt_body,
                             tuple(jnp.zeros((16,), jnp.int32)
                                   for _ in range(E)))
        cnt = [jnp.sum(accs[e]) for e in range(E)]
        nb = [(cnt[e] + (BT - 1)) // BT for e in range(E)]
        cumnb = [jnp.int32(0)]
        for e in range(E):
            cumnb.append(cumnb[-1] + nb[e])
        total_blocks = cumnb[E]
        nb_w = jnp.int32(0)
        off_w = jnp.int32(0)
        for e in range(E):
            nb_w = jnp.where(s == e, nb[e], nb_w)
            off_w = off_w + jnp.where(s > e, nb[e] * BT, 0)
        off_w = pl.multiple_of(off_w, BT)

        @pl.when(s < E)
        def _place():
            def z_body(j, carry):
                seg_tok_v[pl.ds(j * 16, 16)] = jnp.zeros((16,), jnp.int32)
                seg_w_v[pl.ds(j * 16, 16)] = jnp.zeros((16,), jnp.float32)
                return carry

            lax.fori_loop(0, T // 16, z_body, 0)
            cursor = jnp.int32(0)
            for ids_v, w_v, p_hbm in ((id1_v, wa_v, p0_hbm),
                                      (id2_v, wb_v, p1_hbm)):
                def pz_body(j, carry):
                    part_v[pl.ds(j * 16, 16)] = jnp.zeros((16,), jnp.int32)
                    return carry

                lax.fori_loop(0, T // 16, pz_body, 0)

                def p_body(j, cur, ids_v=ids_v, w_v=w_v):
                    v = ids_v[pl.ds(j * 16, 16)]
                    w = w_v[pl.ds(j * 16, 16)]
                    m = v == s
                    mi = m.astype(jnp.int32)
                    rank = plsc.cumsum(mi)
                    lpos = cur + rank - 1
                    tok = j * 16 + lax.iota(jnp.int32, 16)
                    plsc.store_scatter(seg_tok_v, [lpos], tok, mask=m)
                    plsc.store_scatter(seg_w_v, [lpos], w, mask=m)
                    plsc.store_scatter(part_v, [tok], off_w + lpos + 1,
                                       mask=m)
                    return cur + jnp.sum(mi)

                cursor = lax.fori_loop(0, T // 16, p_body, cursor)
                pltpu.sync_copy(part_v, p_hbm.at[s])

            def cp_body(i, carry):
                dst = pl.multiple_of(off_w + i * BT, BT)
                pltpu.sync_copy(seg_tok_v.at[pl.ds(i * BT, BT)],
                                stok_hbm.at[pl.ds(dst, BT)])
                pltpu.sync_copy(seg_w_v.at[pl.ds(i * BT, BT)],
                                sw_hbm.at[pl.ds(dst, BT)])
                return carry

            lax.fori_loop(0, nb_w, cp_body, 0)

        @pl.when(s == E)
        def _tail():
            def z_body(j, carry):
                ztok_v[pl.ds(j * 16, 16)] = jnp.zeros((16,), jnp.int32)
                zw_v[pl.ds(j * 16, 16)] = jnp.zeros((16,), jnp.float32)
                return carry

            lax.fori_loop(0, BT // 16, z_body, 0)

            def zc_body(i, carry):
                dst = pl.multiple_of(i * BT, BT)
                pltpu.sync_copy(ztok_v, stok_hbm.at[pl.ds(dst, BT)])
                pltpu.sync_copy(zw_v, sw_hbm.at[pl.ds(dst, BT)])
                return carry

            lax.fori_loop(total_blocks, NBLK, zc_body, 0)

        @pl.when(s == E + 1)
        def _btab():
            for j in range(48 // 16):
                gvec = j * 16 + lax.iota(jnp.int32, 16)
                ev = jnp.zeros((16,), jnp.int32)
                for e in range(1, E):
                    ev = ev + (gvec >= cumnb[e]).astype(jnp.int32)
                btab_v[0, pl.ds(j * 16, 16)] = ev
                btab_v[1, pl.ds(j * 16, 16)] = jnp.minimum(
                    gvec, total_blocks - 1)
                btab_v[2, pl.ds(j * 16, 16)] = (
                    gvec < total_blocks).astype(jnp.int32)
            pltpu.sync_copy(btab_v, btab_hbm)


_dispatch = pl.kernel(
    _dispatch_body, mesh=_mesh, compiler_params=_sc_params,
    out_type=[
        jax.ShapeDtypeStruct((PAD,), jnp.int32),
        jax.ShapeDtypeStruct((PAD,), jnp.float32),
        jax.ShapeDtypeStruct((E, T), jnp.int32),
        jax.ShapeDtypeStruct((E, T), jnp.int32),
        jax.ShapeDtypeStruct((3, 48), jnp.int32),
    ],
    scratch_types=[
        pltpu.VMEM((T,), jnp.int32),
        pltpu.VMEM((T,), jnp.int32),
        pltpu.VMEM((T,), jnp.float32),
        pltpu.VMEM((T,), jnp.float32),
        pltpu.VMEM((T,), jnp.int32),
        pltpu.VMEM((T,), jnp.float32),
        pltpu.VMEM((T,), jnp.int32),
        pltpu.VMEM((3, 48), jnp.int32),
        pltpu.VMEM((BT,), jnp.int32),
        pltpu.VMEM((BT,), jnp.float32),
    ],
)


_HPAD = PAD // 2
_GROWS = _HPAD // 32
_GCH = 48
_NG = _GROWS // _GCH


def _make_gather(qbase):
    def _gather_body(stok_hbm, flat_hbm, xs_hbm, idx_v, rows0_v, rows1_v,
                     gsem, wsem):
        c = lax.axis_index("c")
        s = lax.axis_index("s")
        base = (s * 2 + c) * _GROWS
        rows = (rows0_v, rows1_v)
        pltpu.sync_copy(stok_hbm.at[pl.ds(qbase + base, _GROWS)], idx_v)

        def fire(k):
            buf = rows[k % 2]
            cps = []
            for g in range(_GCH // 16):
                v = idx_v[pl.ds(k * _GCH + g * 16, 16)]
                for r in range(16):
                    cps.append(pltpu.async_copy(
                        flat_hbm.at[v[r]], buf.at[g * 16 + r], gsem))
            return cps

        gcp = [None] * _NG
        wcp = [None] * _NG
        gcp[0] = fire(0)
        for k in range(_NG):
            for cp in gcp[k]:
                cp.wait()
            if k + 1 < _NG:
                if k >= 1:
                    wcp[k - 1].wait()
                gcp[k + 1] = fire(k + 1)
            wcp[k] = pltpu.async_copy(
                rows[k % 2], xs_hbm.at[pl.ds(base + k * _GCH, _GCH)], wsem)
        if _NG >= 2:
            wcp[_NG - 2].wait()
        wcp[_NG - 1].wait()

    return pl.kernel(
        _gather_body, mesh=_mesh, compiler_params=_sc_params,
        out_type=[jax.ShapeDtypeStruct((_HPAD, H), jnp.float32)],
        scratch_types=[
            pltpu.VMEM((_GROWS,), jnp.int32),
            pltpu.VMEM((_GCH, H), jnp.float32),
            pltpu.VMEM((_GCH, H), jnp.float32),
            pltpu.SemaphoreType.DMA,
            pltpu.SemaphoreType.DMA,
        ],
    )


_gather_a = _make_gather(0)
_gather_b = _make_gather(_HPAD)


_HBLK = NBLK // 2


def _ffn1_body(tab_ref, xs_ref, w1_ref, hmid_ref, w1b_ref, goff):
    g = pl.program_id(0) + goff
    prev = jnp.where(g == goff, jnp.int32(-1),
                     tab_ref[0, jnp.maximum(g - 1, 0)])

    @pl.when(tab_ref[0, g] != prev)
    def _cast():
        w1b_ref[...] = w1_ref[0].astype(jnp.bfloat16)

    @pl.when(tab_ref[2, g] == 1)
    def _():
        h = lax.dot_general(xs_ref[...].astype(jnp.bfloat16), w1b_ref[...],
                            (((1,), (1,)), ((), ())),
                            preferred_element_type=jnp.float32)
        h = h * jax.nn.sigmoid(h)
        hmid_ref[...] = h.astype(jnp.bfloat16)


def _run_ffn1(btab, xs_half, W1, goff):
    import functools
    gs = pltpu.PrefetchScalarGridSpec(
        num_scalar_prefetch=1,
        grid=(_HBLK,),
        in_specs=[
            pl.BlockSpec((BT, H),
                         lambda g, tab: (jnp.clip(tab[1, g + goff] - goff,
                                                  0, _HBLK - 1), 0)),
            pl.BlockSpec((1, F, H), lambda g, tab: (tab[0, g + goff], 0, 0)),
        ],
        out_specs=pl.BlockSpec((BT, F),
                               lambda g, tab: (jnp.clip(tab[1, g + goff]
                                                        - goff,
                                                        0, _HBLK - 1), 0)),
        scratch_shapes=[pltpu.VMEM((F, H), jnp.bfloat16)],
    )
    return pl.pallas_call(
        functools.partial(_ffn1_body, goff=goff),
        grid_spec=gs,
        out_shape=jax.ShapeDtypeStruct((_HPAD, F), jnp.bfloat16),
        compiler_params=pltpu.CompilerParams(
            dimension_semantics=("arbitrary",),
            vmem_limit_bytes=100 * 1024 * 1024),
    )(btab, xs_half, W1)


def _ffn2_body(tab_ref, ha_ref, hb_ref, w2_ref, sw_ref, ys_ref, w2b_ref):
    g = pl.program_id(0)
    prev = jnp.where(g == 0, jnp.int32(-1), tab_ref[0, jnp.maximum(g - 1, 0)])

    @pl.when(tab_ref[0, g] != prev)
    def _cast():
        w2b_ref[...] = w2_ref[0].astype(jnp.bfloat16)

    @pl.when(tab_ref[2, g] == 1)
    def _():
        hm = jnp.where(tab_ref[1, g] < _HBLK, ha_ref[...], hb_ref[...])
        y = lax.dot_general(hm, w2b_ref[...],
                            (((1,), (1,)), ((), ())),
                            preferred_element_type=jnp.float32)
        ys_ref[...] = y * sw_ref[0, 0][:, None]


def _run_ffn2(btab, hmid_a, hmid_b, W2, sw3):
    gs = pltpu.PrefetchScalarGridSpec(
        num_scalar_prefetch=1,
        grid=(NBLK,),
        in_specs=[
            pl.BlockSpec((BT, F),
                         lambda g, tab: (jnp.minimum(tab[1, g], _HBLK - 1),
                                         0)),
            pl.BlockSpec((BT, F),
                         lambda g, tab: (jnp.maximum(tab[1, g] - _HBLK, 0),
                                         0)),
            pl.BlockSpec((1, H, F), lambda g, tab: (tab[0, g], 0, 0)),
            pl.BlockSpec((1, 1, BT), lambda g, tab: (tab[1, g], 0, 0)),
        ],
        out_specs=pl.BlockSpec((BT, H), lambda g, tab: (tab[1, g], 0)),
        scratch_shapes=[pltpu.VMEM((H, F), jnp.bfloat16)],
    )
    return pl.pallas_call(
        _ffn2_body,
        grid_spec=gs,
        out_shape=jax.ShapeDtypeStruct((PAD, H), jnp.float32),
        compiler_params=pltpu.CompilerParams(
            dimension_semantics=("arbitrary",),
            vmem_limit_bytes=100 * 1024 * 1024),
    )(btab, hmid_a, hmid_b, W2, sw3)


_CTOK = T // 32
_CCH = 16
_NC = _CTOK // _CCH


def _combine_body(ys_hbm, p0_hbm, p1_hbm, out_hbm,
                  pt_v, i0_v, i1_v, a0_v, a1_v, b0_v, b1_v,
                  psem, gsem, wsem):
    c = lax.axis_index("c")
    s = lax.axis_index("s")
    base = (s * 2 + c) * _CTOK
    av = (a0_v, a1_v)
    bv = (b0_v, b1_v)
    pc = []
    for e in range(E):
        pc.append(pltpu.async_copy(p0_hbm.at[e, pl.ds(base, _CTOK)],
                                   pt_v.at[0, e], psem))
        pc.append(pltpu.async_copy(p1_hbm.at[e, pl.ds(base, _CTOK)],
                                   pt_v.at[1, e], psem))
    for cp in pc:
        cp.wait()
    for j in range(_NC):
        sl = pl.ds(j * _CCH, _CCH)
        acc0 = pt_v[0, 0, sl]
        acc1 = pt_v[1, 0, sl]
        for e in range(1, E):
            acc0 = acc0 + pt_v[0, e, sl]
            acc1 = acc1 + pt_v[1, e, sl]
        i0_v[j, :] = acc0 - 1
        i1_v[j, :] = acc1 - 1
    gcp = [None] * _NC
    wcp = [None] * _NC

    def issue(k):
        cps = []
        v0 = i0_v[k, :]
        v1 = i1_v[k, :]
        for r in range(_CCH):
            cps.append(pltpu.async_copy(
                ys_hbm.at[v0[r]], av[k % 2].at[r], gsem))
            cps.append(pltpu.async_copy(
                ys_hbm.at[v1[r]], bv[k % 2].at[r], gsem))
        return cps

    gcp[0] = issue(0)
    for k in range(_NC):
        for cp in gcp[k]:
            cp.wait()
        if k + 1 < _NC:
            if k >= 1:
                wcp[k - 1].wait()
            gcp[k + 1] = issue(k + 1)
        a, b = av[k % 2], bv[k % 2]
        for r in range(_CCH):
            def add_body(j, carry, r=r, a=a, b=b):
                sl = pl.ds(j * 16, 16)
                a[r, sl] = a[r, sl] + b[r, sl]
                return carry
            lax.fori_loop(0, H // 16, add_body, 0)
        wcp[k] = pltpu.async_copy(
            a, out_hbm.at[pl.ds(base + k * _CCH, _CCH)], wsem)
    wcp[_NC - 2].wait()
    wcp[_NC - 1].wait()


_combine = pl.kernel(
    _combine_body, mesh=_mesh, compiler_params=_sc_params,
    out_type=[jax.ShapeDtypeStruct((T, H), jnp.float32)],
    scratch_types=[
        pltpu.VMEM((2, E, _CTOK), jnp.int32),
        pltpu.VMEM((_NC, _CCH), jnp.int32),
        pltpu.VMEM((_NC, _CCH), jnp.int32),
        pltpu.VMEM((_CCH, H), jnp.float32),
        pltpu.VMEM((_CCH, H), jnp.float32),
        pltpu.VMEM((_CCH, H), jnp.float32),
        pltpu.VMEM((_CCH, H), jnp.float32),
        pltpu.SemaphoreType.DMA,
        pltpu.SemaphoreType.DMA,
        pltpu.SemaphoreType.DMA,
    ],
)


def kernel(hidden_states, Wr, W1, W2):
    b, s, h = hidden_states.shape
    flat = hidden_states.reshape(-1, h)

    i1, i2, wa, wb = _run_router(flat, Wr)
    stok, sw, parts0, parts1, btab = _dispatch(
        i1.reshape(-1), i2.reshape(-1), wa.reshape(-1), wb.reshape(-1))
    (xs_a,) = _gather_a(stok, flat)
    (xs_b,) = _gather_b(stok, flat)
    hmid_a = _run_ffn1(btab, xs_a, W1, 0)
    hmid_b = _run_ffn1(btab, xs_b, W1, _HBLK)
    ys = _run_ffn2(btab, hmid_a, hmid_b, W2, sw.reshape(NBLK, 1, BT))
    (out,) = _combine(ys, parts0, parts1)
    return out.reshape(b, s, h)


# speedup vs baseline: 1.1833x; 1.1833x over previous
import jax
import jax.numpy as jnp
from jax import lax
from jax.experimental import pallas as pl
from jax.experimental.pallas import tpu as pltpu
from jax.experimental.pallas import tpu_sc as plsc

T = 2048
H = 1024
F = 4096
E = 8
BT = 256
NBLK = T * 2 // BT + E
PAD = NBLK * BT

_mesh = plsc.VectorSubcoreMesh(core_axis_name="c", subcore_axis_name="s")
_sc_params = pltpu.CompilerParams(needs_layout_passes=False)


def _router_body(flat_ref, wr_ref, i1_ref, i2_ref, wa_ref, wb_ref):
    l = lax.dot_general(flat_ref[...], wr_ref[...],
                        (((1,), (1,)), ((), ())),
                        preferred_element_type=jnp.float32)
    lane = lax.broadcasted_iota(jnp.int32, (T, E), 1)
    m1 = jnp.max(l, axis=1, keepdims=True)
    i1 = jnp.min(jnp.where(l >= m1, lane, E), axis=1, keepdims=True)
    l2 = jnp.where(lane == i1, jnp.float32(-1e30), l)
    m2 = jnp.max(l2, axis=1, keepdims=True)
    i2 = jnp.min(jnp.where(l2 >= m2, lane, E), axis=1, keepdims=True)
    wa = 1.0 / (1.0 + jnp.exp(m2 - m1))
    i1_ref[...] = i1
    i2_ref[...] = i2
    wa_ref[...] = wa
    wb_ref[...] = 1.0 - wa


def _run_router(flat, Wr):
    return pl.pallas_call(
        _router_body,
        out_shape=[
            jax.ShapeDtypeStruct((T, 1), jnp.int32),
            jax.ShapeDtypeStruct((T, 1), jnp.int32),
            jax.ShapeDtypeStruct((T, 1), jnp.float32),
            jax.ShapeDtypeStruct((T, 1), jnp.float32),
        ],
    )(flat, Wr)


def _dispatch_body(top1_hbm, top2_hbm, wa_hbm, wb_hbm,
                   stok_hbm, sw_hbm, p0_hbm, p1_hbm, btab_hbm,
                   id1_v, id2_v, wa_v, wb_v, seg_tok_v, seg_w_v, part_v,
                   btab_v, ztok_v, zw_v):
    c = lax.axis_index("c")
    s = lax.axis_index("s")

    @pl.when(jnp.logical_and(c == 0, s < 10))
    def _work():
        pltpu.sync_copy(top1_hbm, id1_v)
        pltpu.sync_copy(top2_hbm, id2_v)
        pltpu.sync_copy(wa_hbm, wa_v)
        pltpu.sync_copy(wb_hbm, wb_v)

        def cnt_body(j, acc):
            v1 = id1_v[pl.ds(j * 16, 16)]
            v2 = id2_v[pl.ds(j * 16, 16)]
            return tuple(acc[e]
                         + (v1 == e).astype(jnp.int32)
                         + (v2 == e).astype(jnp.int32)
                         for e in range(E))

        accs = lax.fori_loop(0, T // 16, cnt_body,
                             tuple(jnp.zeros((16,), jnp.int32)
                                   for _ in range(E)))
        cnt = [jnp.sum(accs[e]) for e in range(E)]
        nb = [(cnt[e] + (BT - 1)) // BT for e in range(E)]
        cumnb = [jnp.int32(0)]
        for e in range(E):
            cumnb.append(cumnb[-1] + nb[e])
        total_blocks = cumnb[E]
        nb_w = jnp.int32(0)
        off_w = jnp.int32(0)
        for e in range(E):
            nb_w = jnp.where(s == e, nb[e], nb_w)
            off_w = off_w + jnp.where(s > e, nb[e] * BT, 0)
        off_w = pl.multiple_of(off_w, BT)

        @pl.when(s < E)
        def _place():
            def z_body(j, carry):
                seg_tok_v[pl.ds(j * 16, 16)] = jnp.zeros((16,), jnp.int32)
                seg_w_v[pl.ds(j * 16, 16)] = jnp.zeros((16,), jnp.float32)
                return carry

            lax.fori_loop(0, T // 16, z_body, 0)
            cursor = jnp.int32(0)
            for ids_v, w_v, p_hbm in ((id1_v, wa_v, p0_hbm),
                                      (id2_v, wb_v, p1_hbm)):
                def pz_body(j, carry):
                    part_v[pl.ds(j * 16, 16)] = jnp.zeros((16,), jnp.int32)
                    return carry

                lax.fori_loop(0, T // 16, pz_body, 0)

                def p_body(j, cur, ids_v=ids_v, w_v=w_v):
                    v = ids_v[pl.ds(j * 16, 16)]
                    w = w_v[pl.ds(j * 16, 16)]
                    m = v == s
                    mi = m.astype(jnp.int32)
                    rank = plsc.cumsum(mi)
                    lpos = cur + rank - 1
                    tok = j * 16 + lax.iota(jnp.int32, 16)
                    plsc.store_scatter(seg_tok_v, [lpos], tok, mask=m)
                    plsc.store_scatter(seg_w_v, [lpos], w, mask=m)
                    plsc.store_scatter(part_v, [tok], off_w + lpos + 1,
                                       mask=m)
                    return cur + jnp.sum(mi)

                cursor = lax.fori_loop(0, T // 16, p_body, cursor)
                pltpu.sync_copy(part_v, p_hbm.at[s])

            def cp_body(i, carry):
                dst = pl.multiple_of(off_w + i * BT, BT)
                pltpu.sync_copy(seg_tok_v.at[pl.ds(i * BT, BT)],
                                stok_hbm.at[pl.ds(dst, BT)])
                pltpu.sync_copy(seg_w_v.at[pl.ds(i * BT, BT)],
                                sw_hbm.at[pl.ds(dst, BT)])
                return carry

            lax.fori_loop(0, nb_w, cp_body, 0)

        @pl.when(s == E)
        def _tail():
            def z_body(j, carry):
                ztok_v[pl.ds(j * 16, 16)] = jnp.zeros((16,), jnp.int32)
                zw_v[pl.ds(j * 16, 16)] = jnp.zeros((16,), jnp.float32)
                return carry

            lax.fori_loop(0, BT // 16, z_body, 0)

            def zc_body(i, carry):
                dst = pl.multiple_of(i * BT, BT)
                pltpu.sync_copy(ztok_v, stok_hbm.at[pl.ds(dst, BT)])
                pltpu.sync_copy(zw_v, sw_hbm.at[pl.ds(dst, BT)])
                return carry

            lax.fori_loop(total_blocks, NBLK, zc_body, 0)

        @pl.when(s == E + 1)
        def _btab():
            for j in range(48 // 16):
                gvec = j * 16 + lax.iota(jnp.int32, 16)
                ev = jnp.zeros((16,), jnp.int32)
                for e in range(1, E):
                    ev = ev + (gvec >= cumnb[e]).astype(jnp.int32)
                btab_v[0, pl.ds(j * 16, 16)] = ev
                btab_v[1, pl.ds(j * 16, 16)] = jnp.minimum(
                    gvec, total_blocks - 1)
                btab_v[2, pl.ds(j * 16, 16)] = (
                    gvec < total_blocks).astype(jnp.int32)
            pltpu.sync_copy(btab_v, btab_hbm)


_dispatch = pl.kernel(
    _dispatch_body, mesh=_mesh, compiler_params=_sc_params,
    out_type=[
        jax.ShapeDtypeStruct((PAD,), jnp.int32),
        jax.ShapeDtypeStruct((PAD,), jnp.float32),
        jax.ShapeDtypeStruct((E, T), jnp.int32),
        jax.ShapeDtypeStruct((E, T), jnp.int32),
        jax.ShapeDtypeStruct((3, 48), jnp.int32),
    ],
    scratch_types=[
        pltpu.VMEM((T,), jnp.int32),
        pltpu.VMEM((T,), jnp.int32),
        pltpu.VMEM((T,), jnp.float32),
        pltpu.VMEM((T,), jnp.float32),
        pltpu.VMEM((T,), jnp.int32),
        pltpu.VMEM((T,), jnp.float32),
        pltpu.VMEM((T,), jnp.int32),
        pltpu.VMEM((3, 48), jnp.int32),
        pltpu.VMEM((BT,), jnp.int32),
        pltpu.VMEM((BT,), jnp.float32),
    ],
)


_GROWS = PAD // 32
_GCH = 48
_NG = _GROWS // _GCH


def _gather_body(stok_hbm, btab_hbm, flat_hbm, xs_hbm, idx_v, btv_v,
                 rows0_v, rows1_v, gsem, wsem):
    c = lax.axis_index("c")
    s = lax.axis_index("s")
    base = (s * 2 + c) * _GROWS
    rows = (rows0_v, rows1_v)
    pltpu.sync_copy(stok_hbm.at[pl.ds(base, _GROWS)], idx_v)
    pltpu.sync_copy(btab_hbm.at[pl.ds(2, 1)], btv_v)
    nact = jnp.int32(0)
    for j in range(48 // 16):
        nact = nact + jnp.sum(btv_v[0, pl.ds(j * 16, 16)])
    rows_total = nact * BT

    def fire(k, buf):
        cps = []
        for g in range(_GCH // 16):
            v = idx_v[pl.ds(k * _GCH + g * 16, 16)]
            for r in range(16):
                cps.append(pltpu.async_copy(
                    flat_hbm.at[v[r]], buf.at[g * 16 + r], gsem))
        return cps

    for k in range(_NG):
        @pl.when(base + k * _GCH < rows_total)
        def _chunk(k=k):
            buf = rows[k % 2]
            cps = fire(k, buf)
            for cp in cps:
                cp.wait()
            pltpu.async_copy(
                buf, xs_hbm.at[pl.ds(base + k * _GCH, _GCH)], wsem).wait()


_gather = pl.kernel(
    _gather_body, mesh=_mesh, compiler_params=_sc_params,
    out_type=[jax.ShapeDtypeStruct((PAD, H), jnp.float32)],
    scratch_types=[
        pltpu.VMEM((_GROWS,), jnp.int32),
        pltpu.VMEM((1, 48), jnp.int32),
        pltpu.VMEM((_GCH, H), jnp.float32),
        pltpu.VMEM((_GCH, H), jnp.float32),
        pltpu.SemaphoreType.DMA,
        pltpu.SemaphoreType.DMA,
    ],
)


def _ffn1_body(tab_ref, xs_ref, w1_ref, hmid_ref, w1b_ref):
    g = pl.program_id(0)
    prev = jnp.where(g == 0, jnp.int32(-1), tab_ref[0, jnp.maximum(g - 1, 0)])

    @pl.when(tab_ref[0, g] != prev)
    def _cast():
        w1b_ref[...] = w1_ref[0].astype(jnp.bfloat16)

    @pl.when(tab_ref[2, g] == 1)
    def _():
        h = lax.dot_general(xs_ref[...].astype(jnp.bfloat16), w1b_ref[...],
                            (((1,), (1,)), ((), ())),
                            preferred_element_type=jnp.float32)
        h = h * jax.nn.sigmoid(h)
        hmid_ref[...] = h.astype(jnp.bfloat16)


def _ffn2_body(tab_ref, hmid_ref, w2_ref, sw_ref, ys_ref, w2b_ref):
    g = pl.program_id(0)
    prev = jnp.where(g == 0, jnp.int32(-1), tab_ref[0, jnp.maximum(g - 1, 0)])

    @pl.when(tab_ref[0, g] != prev)
    def _cast():
        w2b_ref[...] = w2_ref[0].astype(jnp.bfloat16)

    @pl.when(tab_ref[2, g] == 1)
    def _():
        y = lax.dot_general(hmid_ref[...], w2b_ref[...],
                            (((1,), (1,)), ((), ())),
                            preferred_element_type=jnp.float32)
        ys_ref[...] = y * sw_ref[0, 0][:, None]


def _run_ffn(btab, xs, W1, W2, sw3):
    gs1 = pltpu.PrefetchScalarGridSpec(
        num_scalar_prefetch=1,
        grid=(NBLK,),
        in_specs=[
            pl.BlockSpec((BT, H), lambda g, tab: (tab[1, g], 0)),
            pl.BlockSpec((1, F, H), lambda g, tab: (tab[0, g], 0, 0)),
        ],
        out_specs=pl.BlockSpec((BT, F), lambda g, tab: (tab[1, g], 0)),
        scratch_shapes=[pltpu.VMEM((F, H), jnp.bfloat16)],
    )
    hmid = pl.pallas_call(
        _ffn1_body,
        grid_spec=gs1,
        out_shape=jax.ShapeDtypeStruct((PAD, F), jnp.bfloat16),
        compiler_params=pltpu.CompilerParams(
            dimension_semantics=("arbitrary",),
            vmem_limit_bytes=100 * 1024 * 1024),
    )(btab, xs, W1)
    gs2 = pltpu.PrefetchScalarGridSpec(
        num_scalar_prefetch=1,
        grid=(NBLK,),
        in_specs=[
            pl.BlockSpec((BT, F), lambda g, tab: (tab[1, g], 0)),
            pl.BlockSpec((1, H, F), lambda g, tab: (tab[0, g], 0, 0)),
            pl.BlockSpec((1, 1, BT), lambda g, tab: (tab[1, g], 0, 0)),
        ],
        out_specs=pl.BlockSpec((BT, H), lambda g, tab: (tab[1, g], 0)),
        scratch_shapes=[pltpu.VMEM((H, F), jnp.bfloat16)],
    )
    return pl.pallas_call(
        _ffn2_body,
        grid_spec=gs2,
        out_shape=jax.ShapeDtypeStruct((PAD, H), jnp.float32),
        compiler_params=pltpu.CompilerParams(
            dimension_semantics=("arbitrary",),
            vmem_limit_bytes=100 * 1024 * 1024),
    )(btab, hmid, W2, sw3)


_CTOK = T // 32
_CCH = 16
_NC = _CTOK // _CCH


def _combine_body(ys_hbm, p0_hbm, p1_hbm, out_hbm,
                  pt_v, i0_v, i1_v, a0_v, a1_v, b0_v, b1_v,
                  psem, gsem, wsem):
    c = lax.axis_index("c")
    s = lax.axis_index("s")
    base = (s * 2 + c) * _CTOK
    av = (a0_v, a1_v)
    bv = (b0_v, b1_v)
    pc = []
    for e in range(E):
        pc.append(pltpu.async_copy(p0_hbm.at[e, pl.ds(base, _CTOK)],
                                   pt_v.at[0, e], psem))
        pc.append(pltpu.async_copy(p1_hbm.at[e, pl.ds(base, _CTOK)],
                                   pt_v.at[1, e], psem))
    for cp in pc:
        cp.wait()
    for j in range(_NC):
        sl = pl.ds(j * _CCH, _CCH)
        acc0 = pt_v[0, 0, sl]
        acc1 = pt_v[1, 0, sl]
        for e in range(1, E):
            acc0 = acc0 + pt_v[0, e, sl]
            acc1 = acc1 + pt_v[1, e, sl]
        i0_v[j, :] = acc0 - 1
        i1_v[j, :] = acc1 - 1
    gcp = [None] * _NC
    wcp = [None] * _NC

    def issue(k):
        ga = pltpu.async_copy(ys_hbm.at[i0_v.at[k]], av[k % 2], gsem)
        gb = pltpu.async_copy(ys_hbm.at[i1_v.at[k]], bv[k % 2], gsem)
        return (ga, gb)

    gcp[0] = issue(0)
    for k in range(_NC):
        gcp[k][0].wait()
        gcp[k][1].wait()
        if k + 1 < _NC:
            if k >= 1:
                wcp[k - 1].wait()
            gcp[k + 1] = issue(k + 1)
        a, b = av[k % 2], bv[k % 2]
        for r in range(_CCH):
            def add_body(j, carry, r=r, a=a, b=b):
                sl = pl.ds(j * 16, 16)
                a[r, sl] = a[r, sl] + b[r, sl]
                return carry
            lax.fori_loop(0, H // 16, add_body, 0)
        wcp[k] = pltpu.async_copy(
            a, out_hbm.at[pl.ds(base + k * _CCH, _CCH)], wsem)
    wcp[_NC - 2].wait()
    wcp[_NC - 1].wait()


_combine = pl.kernel(
    _combine_body, mesh=_mesh, compiler_params=_sc_params,
    out_type=[jax.ShapeDtypeStruct((T, H), jnp.float32)],
    scratch_types=[
        pltpu.VMEM((2, E, _CTOK), jnp.int32),
        pltpu.VMEM((_NC, _CCH), jnp.int32),
        pltpu.VMEM((_NC, _CCH), jnp.int32),
        pltpu.VMEM((_CCH, H), jnp.float32),
        pltpu.VMEM((_CCH, H), jnp.float32),
        pltpu.VMEM((_CCH, H), jnp.float32),
        pltpu.VMEM((_CCH, H), jnp.float32),
        pltpu.SemaphoreType.DMA,
        pltpu.SemaphoreType.DMA,
        pltpu.SemaphoreType.DMA,
    ],
)


def kernel(hidden_states, Wr, W1, W2):
    b, s, h = hidden_states.shape
    flat = hidden_states.reshape(-1, h)

    i1, i2, wa, wb = _run_router(flat, Wr)
    stok, sw, parts0, parts1, btab = _dispatch(
        i1.reshape(-1), i2.reshape(-1), wa.reshape(-1), wb.reshape(-1))
    (xs,) = _gather(stok, btab, flat)
    ys = _run_ffn(btab, xs, W1, W2, sw.reshape(NBLK, 1, BT))
    (out,) = _combine(ys, parts0, parts1)
    return out.reshape(b, s, h)


# speedup vs baseline: 1.2095x; 1.0222x over previous
import jax
import jax.numpy as jnp
from jax import lax
from jax.experimental import pallas as pl
from jax.experimental.pallas import tpu as pltpu
from jax.experimental.pallas import tpu_sc as plsc

T = 2048
H = 1024
F = 4096
E = 8
BT = 256
NBLK = T * 2 // BT + E
PAD = NBLK * BT

_mesh = plsc.VectorSubcoreMesh(core_axis_name="c", subcore_axis_name="s")
_sc_params = pltpu.CompilerParams(needs_layout_passes=False)


def _router_body(flat_ref, wr_ref, i1_ref, i2_ref, wa_ref, wb_ref):
    l = lax.dot_general(flat_ref[...], wr_ref[...],
                        (((1,), (1,)), ((), ())),
                        preferred_element_type=jnp.float32)
    lane = lax.broadcasted_iota(jnp.int32, (T, E), 1)
    m1 = jnp.max(l, axis=1, keepdims=True)
    i1 = jnp.min(jnp.where(l >= m1, lane, E), axis=1, keepdims=True)
    l2 = jnp.where(lane == i1, jnp.float32(-1e30), l)
    m2 = jnp.max(l2, axis=1, keepdims=True)
    i2 = jnp.min(jnp.where(l2 >= m2, lane, E), axis=1, keepdims=True)
    wa = 1.0 / (1.0 + jnp.exp(m2 - m1))
    i1_ref[...] = i1.reshape(-1)
    i2_ref[...] = i2.reshape(-1)
    wa_ref[...] = wa.reshape(-1)
    wb_ref[...] = (1.0 - wa).reshape(-1)


def _run_router(flat, Wr):
    return pl.pallas_call(
        _router_body,
        out_shape=[
            jax.ShapeDtypeStruct((T,), jnp.int32),
            jax.ShapeDtypeStruct((T,), jnp.int32),
            jax.ShapeDtypeStruct((T,), jnp.float32),
            jax.ShapeDtypeStruct((T,), jnp.float32),
        ],
    )(flat, Wr)


def _dispatch_body(top1_hbm, top2_hbm, wa_hbm, wb_hbm,
                   stok_hbm, sw_hbm, p0_hbm, p1_hbm, btab_hbm,
                   id1_v, id2_v, wa_v, wb_v, seg_tok_v, seg_w_v, part_v,
                   btab_v, ztok_v, zw_v):
    c = lax.axis_index("c")
    s = lax.axis_index("s")

    @pl.when(jnp.logical_and(c == 0, s < 10))
    def _work():
        pltpu.sync_copy(top1_hbm, id1_v)
        pltpu.sync_copy(top2_hbm, id2_v)
        pltpu.sync_copy(wa_hbm, wa_v)
        pltpu.sync_copy(wb_hbm, wb_v)

        def cnt_body(j, acc):
            v1 = id1_v[pl.ds(j * 16, 16)]
            v2 = id2_v[pl.ds(j * 16, 16)]
            return tuple(acc[e]
                         + (v1 == e).astype(jnp.int32)
                         + (v2 == e).astype(jnp.int32)
                         for e in range(E))

        accs = lax.fori_loop(0, T // 16, cnt_body,
                             tuple(jnp.zeros((16,), jnp.int32)
                                   for _ in range(E)))
        cnt = [jnp.sum(accs[e]) for e in range(E)]
        nb = [(cnt[e] + (BT - 1)) // BT for e in range(E)]
        cumnb = [jnp.int32(0)]
        for e in range(E):
            cumnb.append(cumnb[-1] + nb[e])
        total_blocks = cumnb[E]
        nb_w = jnp.int32(0)
        off_w = jnp.int32(0)
        for e in range(E):
            nb_w = jnp.where(s == e, nb[e], nb_w)
            off_w = off_w + jnp.where(s > e, nb[e] * BT, 0)
        off_w = pl.multiple_of(off_w, BT)

        @pl.when(s < E)
        def _place():
            def z_body(j, carry):
                seg_tok_v[pl.ds(j * 16, 16)] = jnp.zeros((16,), jnp.int32)
                seg_w_v[pl.ds(j * 16, 16)] = jnp.zeros((16,), jnp.float32)
                return carry

            lax.fori_loop(0, T // 16, z_body, 0)
            cursor = jnp.int32(0)
            for ids_v, w_v, p_hbm in ((id1_v, wa_v, p0_hbm),
                                      (id2_v, wb_v, p1_hbm)):
                def pz_body(j, carry):
                    part_v[pl.ds(j * 16, 16)] = jnp.zeros((16,), jnp.int32)
                    return carry

                lax.fori_loop(0, T // 16, pz_body, 0)

                def p_body(j, cur, ids_v=ids_v, w_v=w_v):
                    v = ids_v[pl.ds(j * 16, 16)]
                    w = w_v[pl.ds(j * 16, 16)]
                    m = v == s
                    mi = m.astype(jnp.int32)
                    rank = plsc.cumsum(mi)
                    lpos = cur + rank - 1
                    tok = j * 16 + lax.iota(jnp.int32, 16)
                    plsc.store_scatter(seg_tok_v, [lpos], tok, mask=m)
                    plsc.store_scatter(seg_w_v, [lpos], w, mask=m)
                    plsc.store_scatter(part_v, [tok], off_w + lpos + 1,
                                       mask=m)
                    return cur + jnp.sum(mi)

                cursor = lax.fori_loop(0, T // 16, p_body, cursor)
                pltpu.sync_copy(part_v, p_hbm.at[s])

            def cp_body(i, carry):
                dst = pl.multiple_of(off_w + i * BT, BT)
                pltpu.sync_copy(seg_tok_v.at[pl.ds(i * BT, BT)],
                                stok_hbm.at[pl.ds(dst, BT)])
                pltpu.sync_copy(seg_w_v.at[pl.ds(i * BT, BT)],
                                sw_hbm.at[pl.ds(dst, BT)])
                return carry

            lax.fori_loop(0, nb_w, cp_body, 0)

        @pl.when(s == E)
        def _tail():
            def z_body(j, carry):
                ztok_v[pl.ds(j * 16, 16)] = jnp.zeros((16,), jnp.int32)
                zw_v[pl.ds(j * 16, 16)] = jnp.zeros((16,), jnp.float32)
                return carry

            lax.fori_loop(0, BT // 16, z_body, 0)

            def zc_body(i, carry):
                dst = pl.multiple_of(i * BT, BT)
                pltpu.sync_copy(ztok_v, stok_hbm.at[pl.ds(dst, BT)])
                pltpu.sync_copy(zw_v, sw_hbm.at[pl.ds(dst, BT)])
                return carry

            lax.fori_loop(total_blocks, NBLK, zc_body, 0)

        @pl.when(s == E + 1)
        def _btab():
            for j in range(48 // 16):
                gvec = j * 16 + lax.iota(jnp.int32, 16)
                ev = jnp.zeros((16,), jnp.int32)
                for e in range(1, E):
                    ev = ev + (gvec >= cumnb[e]).astype(jnp.int32)
                btab_v[0, pl.ds(j * 16, 16)] = ev
                btab_v[1, pl.ds(j * 16, 16)] = jnp.minimum(
                    gvec, total_blocks - 1)
                btab_v[2, pl.ds(j * 16, 16)] = (
                    gvec < total_blocks).astype(jnp.int32)
            pltpu.sync_copy(btab_v, btab_hbm)


_dispatch = pl.kernel(
    _dispatch_body, mesh=_mesh, compiler_params=_sc_params,
    out_type=[
        jax.ShapeDtypeStruct((PAD,), jnp.int32),
        jax.ShapeDtypeStruct((PAD,), jnp.float32),
        jax.ShapeDtypeStruct((E, T), jnp.int32),
        jax.ShapeDtypeStruct((E, T), jnp.int32),
        jax.ShapeDtypeStruct((3, 48), jnp.int32),
    ],
    scratch_types=[
        pltpu.VMEM((T,), jnp.int32),
        pltpu.VMEM((T,), jnp.int32),
        pltpu.VMEM((T,), jnp.float32),
        pltpu.VMEM((T,), jnp.float32),
        pltpu.VMEM((T,), jnp.int32),
        pltpu.VMEM((T,), jnp.float32),
        pltpu.VMEM((T,), jnp.int32),
        pltpu.VMEM((3, 48), jnp.int32),
        pltpu.VMEM((BT,), jnp.int32),
        pltpu.VMEM((BT,), jnp.float32),
    ],
)


_GROWS = PAD // 32
_GCH = 48
_NG = _GROWS // _GCH


def _gather_body(stok_hbm, btab_hbm, flat_hbm, xs_hbm, idx_v, btv_v,
                 rows0_v, rows1_v, gsem, wsem):
    c = lax.axis_index("c")
    s = lax.axis_index("s")
    base = (s * 2 + c) * _GROWS
    rows = (rows0_v, rows1_v)
    pltpu.sync_copy(stok_hbm.at[pl.ds(base, _GROWS)], idx_v)
    pltpu.sync_copy(btab_hbm.at[pl.ds(2, 1)], btv_v)
    nact = jnp.int32(0)
    for j in range(48 // 16):
        nact = nact + jnp.sum(btv_v[0, pl.ds(j * 16, 16)])
    rows_total = nact * BT

    def fire(k, buf):
        cps = []
        for g in range(_GCH // 16):
            v = idx_v[pl.ds(k * _GCH + g * 16, 16)]
            for r in range(16):
                cps.append(pltpu.async_copy(
                    flat_hbm.at[v[r]], buf.at[g * 16 + r], gsem))
        return cps

    for k in range(_NG):
        @pl.when(base + k * _GCH < rows_total)
        def _chunk(k=k):
            buf = rows[k % 2]
            cps = fire(k, buf)
            for cp in cps:
                cp.wait()
            pltpu.async_copy(
                buf, xs_hbm.at[pl.ds(base + k * _GCH, _GCH)], wsem).wait()


_gather = pl.kernel(
    _gather_body, mesh=_mesh, compiler_params=_sc_params,
    out_type=[jax.ShapeDtypeStruct((PAD, H), jnp.float32)],
    scratch_types=[
        pltpu.VMEM((_GROWS,), jnp.int32),
        pltpu.VMEM((1, 48), jnp.int32),
        pltpu.VMEM((_GCH, H), jnp.float32),
        pltpu.VMEM((_GCH, H), jnp.float32),
        pltpu.SemaphoreType.DMA,
        pltpu.SemaphoreType.DMA,
    ],
)


def _ffn1_body(tab_ref, xs_ref, w1_ref, hmid_ref, w1b_ref):
    g = pl.program_id(0)
    prev = jnp.where(g == 0, jnp.int32(-1), tab_ref[0, jnp.maximum(g - 1, 0)])

    @pl.when(tab_ref[0, g] != prev)
    def _cast():
        w1b_ref[...] = w1_ref[0].astype(jnp.bfloat16)

    @pl.when(tab_ref[2, g] == 1)
    def _():
        h = lax.dot_general(xs_ref[...].astype(jnp.bfloat16), w1b_ref[...],
                            (((1,), (1,)), ((), ())),
                            preferred_element_type=jnp.float32)
        h = h * jax.nn.sigmoid(h)
        hmid_ref[...] = h.astype(jnp.bfloat16)


def _ffn2_body(tab_ref, hmid_ref, w2_ref, sw_ref, ys_ref, w2b_ref):
    g = pl.program_id(0)
    prev = jnp.where(g == 0, jnp.int32(-1), tab_ref[0, jnp.maximum(g - 1, 0)])

    @pl.when(tab_ref[0, g] != prev)
    def _cast():
        w2b_ref[...] = w2_ref[0].astype(jnp.bfloat16)

    @pl.when(tab_ref[2, g] == 1)
    def _():
        y = lax.dot_general(hmid_ref[...], w2b_ref[...],
                            (((1,), (1,)), ((), ())),
                            preferred_element_type=jnp.float32)
        ys_ref[...] = y * sw_ref[0, 0][:, None]


def _run_ffn(btab, xs, W1, W2, sw3):
    gs1 = pltpu.PrefetchScalarGridSpec(
        num_scalar_prefetch=1,
        grid=(NBLK,),
        in_specs=[
            pl.BlockSpec((BT, H), lambda g, tab: (tab[1, g], 0)),
            pl.BlockSpec((1, F, H), lambda g, tab: (tab[0, g], 0, 0)),
        ],
        out_specs=pl.BlockSpec((BT, F), lambda g, tab: (tab[1, g], 0)),
        scratch_shapes=[pltpu.VMEM((F, H), jnp.bfloat16)],
    )
    hmid = pl.pallas_call(
        _ffn1_body,
        grid_spec=gs1,
        out_shape=jax.ShapeDtypeStruct((PAD, F), jnp.bfloat16),
        compiler_params=pltpu.CompilerParams(
            dimension_semantics=("arbitrary",),
            vmem_limit_bytes=100 * 1024 * 1024),
    )(btab, xs, W1)
    gs2 = pltpu.PrefetchScalarGridSpec(
        num_scalar_prefetch=1,
        grid=(NBLK,),
        in_specs=[
            pl.BlockSpec((BT, F), lambda g, tab: (tab[1, g], 0)),
            pl.BlockSpec((1, H, F), lambda g, tab: (tab[0, g], 0, 0)),
            pl.BlockSpec((1, 1, BT), lambda g, tab: (tab[1, g], 0, 0)),
        ],
        out_specs=pl.BlockSpec((BT, H), lambda g, tab: (tab[1, g], 0)),
        scratch_shapes=[pltpu.VMEM((H, F), jnp.bfloat16)],
    )
    return pl.pallas_call(
        _ffn2_body,
        grid_spec=gs2,
        out_shape=jax.ShapeDtypeStruct((PAD, H), jnp.float32),
        compiler_params=pltpu.CompilerParams(
            dimension_semantics=("arbitrary",),
            vmem_limit_bytes=100 * 1024 * 1024),
    )(btab, hmid, W2, sw3)


_CTOK = T // 32
_CCH = 16
_NC = _CTOK // _CCH


def _combine_body(ys_hbm, p0_hbm, p1_hbm, out_hbm,
                  pt_v, i0_v, i1_v, a0_v, a1_v, b0_v, b1_v,
                  psem, gsem, wsem):
    c = lax.axis_index("c")
    s = lax.axis_index("s")
    base = (s * 2 + c) * _CTOK
    av = (a0_v, a1_v)
    bv = (b0_v, b1_v)
    pc = []
    for e in range(E):
        pc.append(pltpu.async_copy(p0_hbm.at[e, pl.ds(base, _CTOK)],
                                   pt_v.at[0, e], psem))
        pc.append(pltpu.async_copy(p1_hbm.at[e, pl.ds(base, _CTOK)],
                                   pt_v.at[1, e], psem))
    for cp in pc:
        cp.wait()
    for j in range(_NC):
        sl = pl.ds(j * _CCH, _CCH)
        acc0 = pt_v[0, 0, sl]
        acc1 = pt_v[1, 0, sl]
        for e in range(1, E):
            acc0 = acc0 + pt_v[0, e, sl]
            acc1 = acc1 + pt_v[1, e, sl]
        i0_v[j, :] = acc0 - 1
        i1_v[j, :] = acc1 - 1
    gcp = [None] * _NC
    wcp = [None] * _NC

    def issue(k):
        ga = pltpu.async_copy(ys_hbm.at[i0_v.at[k]], av[k % 2], gsem)
        gb = pltpu.async_copy(ys_hbm.at[i1_v.at[k]], bv[k % 2], gsem)
        return (ga, gb)

    gcp[0] = issue(0)
    for k in range(_NC):
        gcp[k][0].wait()
        gcp[k][1].wait()
        if k + 1 < _NC:
            if k >= 1:
                wcp[k - 1].wait()
            gcp[k + 1] = issue(k + 1)
        a, b = av[k % 2], bv[k % 2]
        for r in range(_CCH):
            def add_body(j, carry, r=r, a=a, b=b):
                sl = pl.ds(j * 16, 16)
                a[r, sl] = a[r, sl] + b[r, sl]
                return carry
            lax.fori_loop(0, H // 16, add_body, 0)
        wcp[k] = pltpu.async_copy(
            a, out_hbm.at[pl.ds(base + k * _CCH, _CCH)], wsem)
    wcp[_NC - 2].wait()
    wcp[_NC - 1].wait()


_combine = pl.kernel(
    _combine_body, mesh=_mesh, compiler_params=_sc_params,
    out_type=[jax.ShapeDtypeStruct((T, H), jnp.float32)],
    scratch_types=[
        pltpu.VMEM((2, E, _CTOK), jnp.int32),
        pltpu.VMEM((_NC, _CCH), jnp.int32),
        pltpu.VMEM((_NC, _CCH), jnp.int32),
        pltpu.VMEM((_CCH, H), jnp.float32),
        pltpu.VMEM((_CCH, H), jnp.float32),
        pltpu.VMEM((_CCH, H), jnp.float32),
        pltpu.VMEM((_CCH, H), jnp.float32),
        pltpu.SemaphoreType.DMA,
        pltpu.SemaphoreType.DMA,
        pltpu.SemaphoreType.DMA,
    ],
)


def kernel(hidden_states, Wr, W1, W2):
    b, s, h = hidden_states.shape
    flat = hidden_states.reshape(-1, h)

    i1, i2, wa, wb = _run_router(flat, Wr)
    stok, sw, parts0, parts1, btab = _dispatch(i1, i2, wa, wb)
    (xs,) = _gather(stok, btab, flat)
    ys = _run_ffn(btab, xs, W1, W2, sw.reshape(NBLK, 1, BT))
    (out,) = _combine(ys, parts0, parts1)
    return out.reshape(b, s, h)
